# baseline (speedup 1.0000x reference)
import functools

import jax
import jax.numpy as jnp
from jax import lax
from jax.experimental import pallas as pl
from jax.experimental.pallas import tpu as pltpu
from jax.experimental.pallas import tpu_sc as plsc

_NCH = 85
_NG = 128
_PLANE = _NG * _NG


def _sc_body(img_hbm, raw_hbm, out_hbm, img_v, in_v, in2_v, out_v):
    nb = raw_hbm.shape[0]
    n_units = nb * 3 * _NCH
    wid = lax.axis_index("s") * 2 + lax.axis_index("c")
    nw = 32
    pltpu.sync_copy(img_hbm, img_v)
    img = img_v[pl.ds(0, 16)]

    def unit_body(i, _):
        u = wid + nw * i

        @pl.when(u < n_units)
        def _run():
            b = u // (3 * _NCH)
            ch = u % (3 * _NCH)
            a = ch // _NCH
            c = ch % _NCH
            pltpu.sync_copy(raw_hbm.at[b, ch], in_v)

            aw = jnp.where(a == 1, 16.0, jnp.where(a == 2, 33.0, 10.0))
            ah = jnp.where(a == 1, 30.0, jnp.where(a == 2, 23.0, 13.0))

            @pl.when(c >= 4)
            def _sigmoid():
                def row(y, _):
                    for x0 in range(_NG // 16):
                        v = in_v[y, pl.ds(x0 * 16, 16)]
                        s = 1.0 / (1.0 + jnp.exp(-v))
                        out_v[pl.ds(y * _NG + x0 * 16, 16)] = s
                    return ()
                lax.fori_loop(0, _NG, row, ())

            @pl.when(c < 4)
            def _box():
                ch2 = jnp.where(c < 2, ch + 2, ch - 2)
                pltpu.sync_copy(raw_hbm.at[b, ch2], in2_v)
                an = jnp.where(c % 2 == 0, aw, ah)

                def row(y, _):
                    yf = y.astype(jnp.float32)
                    for x0 in range(_NG // 16):
                        v = in_v[y, pl.ds(x0 * 16, 16)]
                        v2 = in2_v[y, pl.ds(x0 * 16, 16)]
                        e1 = jnp.minimum(jnp.maximum(jnp.exp(v) * an, 0.0), img)
                        e2 = jnp.minimum(jnp.maximum(jnp.exp(v2) * an, 0.0), img)
                        xs = (lax.iota(jnp.int32, 16) + x0 * 16).astype(jnp.float32)
                        gx = (xs + 0.5) * 4.0
                        gy = (yf + 0.5) * 4.0
                        grid = jnp.where(c == 0, gx, gy)
                        res = jnp.where(c < 2,
                                        grid + (e2 - e1) * 0.5,
                                        e1 + e2)
                        out_v[pl.ds(y * _NG + x0 * 16, 16)] = res
                    return ()
                lax.fori_loop(0, _NG, row, ())

            pltpu.sync_copy(out_v, out_hbm.at[c, b, pl.ds(a * _PLANE, _PLANE)])

        return ()

    lax.fori_loop(0, (n_units + nw - 1) // nw, unit_body, ())


def kernel(raw, img_size):
    nB = raw.shape[0]
    nG = raw.shape[2]
    img = jnp.full((16,), img_size, dtype=jnp.float32)
    mesh = plsc.VectorSubcoreMesh(core_axis_name="c", subcore_axis_name="s")
    run = functools.partial(
        pl.kernel,
        mesh=mesh,
        out_type=jax.ShapeDtypeStruct((_NCH, nB, 3 * nG * nG), jnp.float32),
        scratch_types=[
            pltpu.VMEM((16,), jnp.float32),
            pltpu.VMEM((nG, nG), jnp.float32),
            pltpu.VMEM((nG, nG), jnp.float32),
            pltpu.VMEM((nG * nG,), jnp.float32),
        ],
    )(_sc_body)
    out = run(img, raw)
    return out.transpose(1, 2, 0)


# speedup vs baseline: 3.3135x; 3.3135x over previous
import jax
import jax.numpy as jnp
from jax.experimental import pallas as pl
from jax.experimental.pallas import tpu as pltpu

_NCH = 85
_CG = 5


def _decode_kernel(img_ref, in_ref, out_ref):
    img_size = img_ref[0]
    g = pl.program_id(0)
    a = pl.program_id(1)
    aw = jnp.where(a == 1, 16.0, jnp.where(a == 2, 33.0, 10.0))
    ah = jnp.where(a == 1, 30.0, jnp.where(a == 2, 23.0, 13.0))
    v = in_ref[...]
    nb, _, ng, _ = v.shape
    sig = 0.5 * jnp.tanh(0.5 * v) + 0.5
    out_ref[...] = jnp.transpose(sig, (1, 0, 2, 3)).reshape(_CG, nb, ng * ng)

    @pl.when(g == 0)
    def _decode_boxes():
        ex = jnp.exp(v[:, 0:4])
        anc = jnp.where(
            jax.lax.broadcasted_iota(jnp.int32, (1, 4, 1, 1), 1) % 2 == 0,
            aw, ah)
        e = jnp.clip(ex * anc, 0.0, img_size)
        l = e[:, 0:1]
        t = e[:, 1:2]
        r = e[:, 2:3]
        b = e[:, 3:4]
        gx = jax.lax.broadcasted_iota(
            jnp.int32, (1, 1, ng, ng), 3).astype(jnp.float32)
        gy = jax.lax.broadcasted_iota(
            jnp.int32, (1, 1, ng, ng), 2).astype(jnp.float32)
        cx = (gx + 0.5) * 4.0 + (r - l) * 0.5
        cy = (gy + 0.5) * 4.0 + (b - t) * 0.5
        w = l + r
        h = t + b
        xy = jnp.concatenate([cx, cy, w, h], axis=1)
        out_ref[0:4] = jnp.transpose(xy, (1, 0, 2, 3)).reshape(4, nb, ng * ng)


def kernel(raw, img_size):
    nB = raw.shape[0]
    nG = raw.shape[2]
    nA = 3
    nCH = _NCH
    img = jnp.asarray(img_size, dtype=jnp.float32).reshape(1)
    ngrp = nCH // _CG
    grid = (ngrp, nA)
    out = pl.pallas_call(
        _decode_kernel,
        grid=grid,
        in_specs=[
            pl.BlockSpec(memory_space=pltpu.SMEM),
            pl.BlockSpec((nB, _CG, nG, nG), lambda g, a: (0, ngrp * a + g, 0, 0)),
        ],
        out_specs=pl.BlockSpec((_CG, nB, nG * nG), lambda g, a: (g, 0, a)),
        out_shape=jax.ShapeDtypeStruct((nCH, nB, nA * nG * nG), jnp.float32),
    )(img, raw)
    return out.transpose(1, 2, 0)
